# baseline (speedup 1.0000x reference)
import functools
import math

import jax
import jax.numpy as jnp
import numpy as np
from jax import lax
from jax.experimental import pallas as pl
from jax.experimental.pallas import tpu as pltpu
from jax.experimental.pallas import tpu_sc as plsc

_NC = 2
_NS = 16
_NW = _NC * _NS
_CHUNK = 128
_GATHER_DNUMS = lax.GatherDimensionNumbers(
    offset_dims=(), collapsed_slice_dims=(0,), start_index_map=(0,)
)


def _positional_encoding_np(L, d_model):
    position = np.arange(0, L, dtype=np.float32)[:, None]
    div_term = np.exp(
        np.arange(0, d_model, 2).astype(np.float32) * (-math.log(10000.0) / d_model)
    )
    pe = np.zeros((L, d_model), dtype=np.float32)
    pe[:, 0::2] = np.sin(position * div_term)
    pe[:, 1::2] = np.cos(position * div_term)
    return pe


@functools.lru_cache(maxsize=None)
def _make_gather(n_rows, emb, n_tables, n_tok):
    rows_per_w = n_rows // _NW
    n_chunks = rows_per_w // _CHUNK
    mesh = plsc.VectorSubcoreMesh(core_axis_name="c", subcore_axis_name="s")

    @functools.partial(
        pl.kernel,
        mesh=mesh,
        compiler_params=pltpu.CompilerParams(use_tc_tiling_on_sc=False),
        out_type=[jax.ShapeDtypeStruct((n_rows, 2 * emb), jnp.float32)] * 2,
        scratch_types=[
            pltpu.VMEM((rows_per_w * n_tables,), jnp.int32),
            [pltpu.VMEM((n_chunks, _CHUNK), jnp.int32) for _ in range(n_tables)],
            pltpu.VMEM((rows_per_w, emb), jnp.float32),
            pltpu.SemaphoreType.DMA,
            pltpu.SemaphoreType.DMA,
        ],
    )
    def gather_kernel(
        xf, ta, tb, tc_, td, out_ab, out_cd, xblk, idxs, rows_v, si, sg
    ):
        tabs = (ta, tb, tc_, td)
        outs = (out_ab, out_ab, out_cd, out_cd)
        wid = lax.axis_index("s") * _NC + lax.axis_index("c")
        base = wid * rows_per_w
        pltpu.sync_copy(xf.at[pl.ds(base * n_tables, rows_per_w * n_tables)], xblk)
        lane = lax.iota(jnp.int32, 16)
        perm = (lane % 4) * 4
        in_lo = lane < 8
        lo_half = lane < 4
        hi_half = (lane >= 8) & (lane < 12)
        for g in range(rows_per_w // 16):
            vs = [xblk[pl.ds(g * 64 + 16 * i, 16)] for i in range(4)]
            for t in range(n_tables):
                p = perm + t
                gs = [
                    lax.gather(
                        v,
                        p[:, None],
                        _GATHER_DNUMS,
                        slice_sizes=(1,),
                        mode=lax.GatherScatterMode.PROMISE_IN_BOUNDS,
                    )
                    for v in vs
                ]
                lo = jnp.where(lo_half, gs[0], gs[1])
                hi = jnp.where(hi_half, gs[2], gs[3])
                vec = jnp.where(in_lo, lo, hi)
                j = vec & 8191
                vec = (vec - j) + 2 * (j & 4095) + (j >> 12)
                idxs[t][g // 8, pl.ds((g % 8) * 16, 16)] = vec
        for t in range(n_tables):
            cps = [
                pltpu.async_copy(
                    tabs[t].at[idxs[t].at[c]],
                    rows_v.at[pl.ds(c * _CHUNK, _CHUNK)],
                    sg,
                )
                for c in range(n_chunks)
            ]
            for cp in cps:
                cp.wait()
            pltpu.sync_copy(
                rows_v,
                outs[t].at[pl.ds(base, rows_per_w), pl.ds((t % 2) * emb, emb)],
            )

    return gather_kernel


def _make_depad(n_tok, emb, BW):
    nb = -(-n_tok // BW)

    def body(t_ref, o_ref):
        r = lax.broadcasted_iota(jnp.int32, (emb, emb), 0)
        c = lax.broadcasted_iota(jnp.int32, (emb, emb), 1)
        eye = (r == c).astype(jnp.float32)
        full = lax.dot_general(
            t_ref[...], eye, (((0,), (0,)), ((), ())),
            preferred_element_type=jnp.float32,
        )
        o_ref[:, 0:emb] = full[0 : BW // 2, :]
        o_ref[:, emb : 2 * emb] = full[BW // 2 : BW, :]

    return pl.pallas_call(
        body,
        grid=(nb,),
        in_specs=[pl.BlockSpec((emb, BW), lambda i: (0, i))],
        out_specs=pl.BlockSpec((BW // 2, 2 * emb), lambda i: (i, 0)),
        out_shape=jax.ShapeDtypeStruct((nb * BW // 2, 2 * emb), jnp.float32),
    )


def _make_matmul(n_rows, emb, d_model, L, TL, scale):
    lb = L // TL
    nb = n_rows // L
    grid = (lb, nb)

    def body(eab, ecd, w_ref, b_ref, pe_ref, o_ref):
        acc = jnp.dot(eab[...], w_ref[0 : 2 * emb, :],
                      preferred_element_type=jnp.float32)
        acc += jnp.dot(ecd[...], w_ref[2 * emb : 4 * emb, :],
                       preferred_element_type=jnp.float32)
        o_ref[...] = acc * scale + b_ref[...] + pe_ref[...]

    emb_spec = pl.BlockSpec((TL, 2 * emb), lambda l, b: (b * lb + l, 0))
    return pl.pallas_call(
        body,
        grid=grid,
        in_specs=[
            emb_spec,
            emb_spec,
            pl.BlockSpec((4 * emb, d_model), lambda l, b: (0, 0)),
            pl.BlockSpec((1, d_model), lambda l, b: (0, 0)),
            pl.BlockSpec((TL, d_model), lambda l, b: (l, 0)),
        ],
        out_specs=pl.BlockSpec((TL, d_model), lambda l, b: (b * lb + l, 0)),
        out_shape=jax.ShapeDtypeStruct((n_rows, d_model), jnp.float32),
    )


def kernel(x, table_a, table_b, table_c, table_d, W, b):
    B, L, A = x.shape
    n_tok, emb = table_a.shape
    d_model = W.shape[1]
    n_rows = B * L
    scale = math.sqrt(float(emb))

    xf = x.astype(jnp.int32).reshape(n_rows * A)
    depad = _make_depad(n_tok, emb, 8192)
    nb = -(-n_tok // 8192)
    t2 = [
        depad(t.T).reshape(nb * 8192, emb)
        for t in (table_a, table_b, table_c, table_d)
    ]
    gather = _make_gather(n_rows, emb, A, n_tok)
    eab, ecd = gather(xf, *t2)

    pe = jnp.asarray(_positional_encoding_np(L, d_model))
    matmul = _make_matmul(n_rows, emb, d_model, L, 2048, scale)
    out = matmul(eab, ecd, W, b.reshape(1, d_model), pe)
    return out.reshape(B, L, d_model)


# speedup vs baseline: 1.0051x; 1.0051x over previous
import functools
import math

import jax
import jax.numpy as jnp
import numpy as np
from jax import lax
from jax.experimental import pallas as pl
from jax.experimental.pallas import tpu as pltpu
from jax.experimental.pallas import tpu_sc as plsc

_NC = 2
_NS = 16
_NW = _NC * _NS
_CHUNK = 128
_GATHER_DNUMS = lax.GatherDimensionNumbers(
    offset_dims=(), collapsed_slice_dims=(0,), start_index_map=(0,)
)


def _positional_encoding_np(L, d_model):
    position = np.arange(0, L, dtype=np.float32)[:, None]
    div_term = np.exp(
        np.arange(0, d_model, 2).astype(np.float32) * (-math.log(10000.0) / d_model)
    )
    pe = np.zeros((L, d_model), dtype=np.float32)
    pe[:, 0::2] = np.sin(position * div_term)
    pe[:, 1::2] = np.cos(position * div_term)
    return pe


@functools.lru_cache(maxsize=None)
def _make_gather(n_rows, emb, n_tables, n_tok):
    rows_per_w = n_rows // _NW
    n_chunks = rows_per_w // _CHUNK
    mesh = plsc.VectorSubcoreMesh(core_axis_name="c", subcore_axis_name="s")

    @functools.partial(
        pl.kernel,
        mesh=mesh,
        compiler_params=pltpu.CompilerParams(use_tc_tiling_on_sc=False),
        out_type=[jax.ShapeDtypeStruct((n_rows, 2 * emb), jnp.float32)] * 2,
        scratch_types=[
            pltpu.VMEM((rows_per_w * n_tables,), jnp.int32),
            [pltpu.VMEM((n_chunks, _CHUNK), jnp.int32) for _ in range(n_tables)],
            pltpu.VMEM((rows_per_w, emb), jnp.float32),
            pltpu.SemaphoreType.DMA,
            pltpu.SemaphoreType.DMA,
        ],
    )
    def gather_kernel(
        xf, ta, tb, tc_, td, out_ab, out_cd, xblk, idxs, rows_v, si, sg
    ):
        tabs = (ta, tb, tc_, td)
        outs = (out_ab, out_ab, out_cd, out_cd)
        wid = lax.axis_index("s") * _NC + lax.axis_index("c")
        base = wid * rows_per_w
        pltpu.sync_copy(xf.at[pl.ds(base * n_tables, rows_per_w * n_tables)], xblk)
        lane = lax.iota(jnp.int32, 16)
        perm = (lane % 4) * 4
        in_lo = lane < 8
        lo_half = lane < 4
        hi_half = (lane >= 8) & (lane < 12)
        for g in range(rows_per_w // 16):
            vs = [xblk[pl.ds(g * 64 + 16 * i, 16)] for i in range(4)]
            for t in range(n_tables):
                p = perm + t
                gs = [
                    lax.gather(
                        v,
                        p[:, None],
                        _GATHER_DNUMS,
                        slice_sizes=(1,),
                        mode=lax.GatherScatterMode.PROMISE_IN_BOUNDS,
                    )
                    for v in vs
                ]
                lo = jnp.where(lo_half, gs[0], gs[1])
                hi = jnp.where(hi_half, gs[2], gs[3])
                vec = jnp.where(in_lo, lo, hi)
                j = vec & 8191
                vec = (vec - j) + 2 * (j & 4095) + (j >> 12)
                idxs[t][g // 8, pl.ds((g % 8) * 16, 16)] = vec
        for t in range(n_tables):
            cps = [
                pltpu.async_copy(
                    tabs[t].at[idxs[t].at[c]],
                    rows_v.at[pl.ds(c * _CHUNK, _CHUNK)],
                    sg,
                )
                for c in range(n_chunks)
            ]
            for cp in cps:
                cp.wait()
            pltpu.sync_copy(
                rows_v,
                outs[t].at[pl.ds(base, rows_per_w), pl.ds((t % 2) * emb, emb)],
            )

    return gather_kernel


def _make_depad(n_tok, emb, BW):
    nb = -(-n_tok // BW)

    def body(t_ref, o_ref):
        full = jnp.transpose(t_ref[...])
        o_ref[:, 0:emb] = full[0 : BW // 2, :]
        o_ref[:, emb : 2 * emb] = full[BW // 2 : BW, :]

    return pl.pallas_call(
        body,
        grid=(nb,),
        in_specs=[pl.BlockSpec((emb, BW), lambda i: (0, i))],
        out_specs=pl.BlockSpec((BW // 2, 2 * emb), lambda i: (i, 0)),
        out_shape=jax.ShapeDtypeStruct((nb * BW // 2, 2 * emb), jnp.float32),
    )


def _make_matmul(n_rows, emb, d_model, L, TL, scale):
    lb = L // TL
    nb = n_rows // L
    grid = (lb, nb)

    def body(eab, ecd, w_ref, b_ref, pe_ref, o_ref):
        acc = jnp.dot(eab[...], w_ref[0 : 2 * emb, :],
                      preferred_element_type=jnp.float32)
        acc += jnp.dot(ecd[...], w_ref[2 * emb : 4 * emb, :],
                       preferred_element_type=jnp.float32)
        o_ref[...] = acc * scale + b_ref[...] + pe_ref[...]

    emb_spec = pl.BlockSpec((TL, 2 * emb), lambda l, b: (b * lb + l, 0))
    return pl.pallas_call(
        body,
        grid=grid,
        in_specs=[
            emb_spec,
            emb_spec,
            pl.BlockSpec((4 * emb, d_model), lambda l, b: (0, 0)),
            pl.BlockSpec((1, d_model), lambda l, b: (0, 0)),
            pl.BlockSpec((TL, d_model), lambda l, b: (l, 0)),
        ],
        out_specs=pl.BlockSpec((TL, d_model), lambda l, b: (b * lb + l, 0)),
        out_shape=jax.ShapeDtypeStruct((n_rows, d_model), jnp.float32),
    )


def kernel(x, table_a, table_b, table_c, table_d, W, b):
    B, L, A = x.shape
    n_tok, emb = table_a.shape
    d_model = W.shape[1]
    n_rows = B * L
    scale = math.sqrt(float(emb))

    xf = x.astype(jnp.int32).reshape(n_rows * A)
    depad = _make_depad(n_tok, emb, 8192)
    nb = -(-n_tok // 8192)
    t2 = [
        depad(t.T).reshape(nb * 8192, emb)
        for t in (table_a, table_b, table_c, table_d)
    ]
    gather = _make_gather(n_rows, emb, A, n_tok)
    eab, ecd = gather(xf, *t2)

    pe = jnp.asarray(_positional_encoding_np(L, d_model))
    matmul = _make_matmul(n_rows, emb, d_model, L, 2048, scale)
    out = matmul(eab, ecd, W, b.reshape(1, d_model), pe)
    return out.reshape(B, L, d_model)
